# baseline (speedup 1.0000x reference)
import functools

import jax
import jax.numpy as jnp
from jax import lax
from jax.experimental import pallas as pl
from jax.experimental.pallas import tpu as pltpu
from jax.experimental.pallas import tpu_sc as plsc

_NUM_CORES = 2
_NUM_SUBCORES = 16
_NUM_WORKERS = _NUM_CORES * _NUM_SUBCORES
_CHUNK = 128
_PREP_BLK = 4096
_OUT_BLK = 16


def _prep_body(i_ref, o_ref):
    t = i_ref[...]
    o_ref[:, : t.shape[0]] = t.T
    o_ref[:, t.shape[0] :] = jnp.zeros(
        (t.shape[1], 128 - t.shape[0]), t.dtype
    )


def kernel(indices, weight):
    batch, seq = indices.shape
    vocab, dim = weight.shape
    n = batch * seq
    per_worker = n // _NUM_WORKERS
    n_chunks = per_worker // _CHUNK

    flat_idx = indices.reshape(1, n).astype(jnp.int32)

    w_pad = pl.pallas_call(
        _prep_body,
        grid=((vocab + _PREP_BLK - 1) // _PREP_BLK,),
        in_specs=[pl.BlockSpec((dim, _PREP_BLK), lambda i: (0, i))],
        out_specs=pl.BlockSpec((_PREP_BLK, 128), lambda i: (i, 0)),
        out_shape=jax.ShapeDtypeStruct((vocab, 128), weight.dtype),
    )(weight.T)

    mesh = plsc.VectorSubcoreMesh(core_axis_name="c", subcore_axis_name="s")

    @functools.partial(
        pl.kernel,
        out_type=jax.ShapeDtypeStruct((n, 128), weight.dtype),
        mesh=mesh,
        scratch_types=[
            pltpu.VMEM((per_worker,), jnp.int32),
            pltpu.VMEM((2, _CHUNK, 128), jnp.float32),
            pltpu.SemaphoreType.DMA,
            pltpu.SemaphoreType.DMA,
            pltpu.SemaphoreType.DMA,
            pltpu.SemaphoreType.DMA,
        ],
    )
    def gather_kernel(w_hbm, i_hbm, o_hbm, idx_v, g_v, gs0, gs1, ws0, ws1):
        gsem = (gs0, gs1)
        wsem = (ws0, ws1)

        wid = lax.axis_index("s") * _NUM_CORES + lax.axis_index("c")
        base = wid * per_worker
        pltpu.sync_copy(i_hbm.at[0, pl.ds(base, per_worker)], idx_v)

        def start_gather(slot, c):
            pltpu.async_copy(
                w_hbm.at[idx_v.at[pl.ds(c * _CHUNK, _CHUNK)]],
                g_v.at[slot],
                gsem[slot],
            )

        def wait_gather(slot, c):
            pltpu.make_async_copy(
                w_hbm.at[idx_v.at[pl.ds(c * _CHUNK, _CHUNK)]],
                g_v.at[slot],
                gsem[slot],
            ).wait()

        def start_write(slot, c):
            pltpu.async_copy(
                g_v.at[slot],
                o_hbm.at[pl.ds(base + c * _CHUNK, _CHUNK)],
                wsem[slot],
            )

        def wait_write(slot, c):
            pltpu.make_async_copy(
                g_v.at[slot],
                o_hbm.at[pl.ds(base + c * _CHUNK, _CHUNK)],
                wsem[slot],
            ).wait()

        start_gather(0, 0)
        start_gather(1, 1)

        @pl.loop(0, n_chunks, step=2)
        def _(c):
            for b in range(2):
                cc = c + b
                wait_gather(b, cc)
                start_write(b, cc)

                @pl.when(cc + 2 < n_chunks)
                def _():
                    wait_write(b, cc)
                    start_gather(b, cc + 2)

        wait_write(0, n_chunks - 2)
        wait_write(1, n_chunks - 1)

    rows = gather_kernel(w_pad, flat_idx)

    def _unpack_body(i_ref, o_ref):
        x = i_ref[...]
        o_ref[...] = x[:, :dim].reshape(_OUT_BLK, seq, dim)

    out = pl.pallas_call(
        _unpack_body,
        grid=(batch // _OUT_BLK,),
        in_specs=[pl.BlockSpec((_OUT_BLK * seq, 128), lambda i: (i, 0))],
        out_specs=pl.BlockSpec((_OUT_BLK, seq, dim), lambda i: (i, 0, 0)),
        out_shape=jax.ShapeDtypeStruct((batch, seq, dim), weight.dtype),
    )(rows)

    return out


# speedup vs baseline: 1.5500x; 1.5500x over previous
import functools

import jax
import jax.numpy as jnp
from jax import lax
from jax.experimental import pallas as pl
from jax.experimental.pallas import tpu as pltpu
from jax.experimental.pallas import tpu_sc as plsc

_NUM_CORES = 2
_NUM_SUBCORES = 16
_NUM_WORKERS = _NUM_CORES * _NUM_SUBCORES
_CHUNK = 128
_PREP_BLK = 4096


def kernel(indices, weight):
    batch, seq = indices.shape
    vocab, dim = weight.shape
    n = batch * seq
    per_worker = n // _NUM_WORKERS
    n_chunks = per_worker // _CHUNK

    flat_idx = indices.reshape(1, n).astype(jnp.int32)

    eye = jnp.eye(dim, dtype=weight.dtype)

    def _prep_body(i_ref, e_ref, o_ref):
        x = i_ref[...]
        o_ref[:, :dim] = jax.lax.dot_general(
            x,
            e_ref[...],
            (((0,), (0,)), ((), ())),
            preferred_element_type=jnp.float32,
        )
        o_ref[:, dim:] = jnp.zeros((x.shape[1], 128 - dim), x.dtype)

    w_pad = pl.pallas_call(
        _prep_body,
        grid=((vocab + _PREP_BLK - 1) // _PREP_BLK,),
        in_specs=[
            pl.BlockSpec((dim, _PREP_BLK), lambda i: (0, i)),
            pl.BlockSpec((dim, dim), lambda i: (0, 0)),
        ],
        out_specs=pl.BlockSpec((_PREP_BLK, 128), lambda i: (i, 0)),
        out_shape=jax.ShapeDtypeStruct((vocab, 128), weight.dtype),
    )(weight.T, eye)

    mesh = plsc.VectorSubcoreMesh(core_axis_name="c", subcore_axis_name="s")

    @functools.partial(
        pl.kernel,
        out_type=jax.ShapeDtypeStruct((n, dim), weight.dtype),
        mesh=mesh,
        scratch_types=[
            pltpu.VMEM((per_worker,), jnp.int32),
            pltpu.VMEM((2, _CHUNK, 128), jnp.float32),
            pltpu.VMEM((2, _CHUNK, dim), jnp.float32),
            pltpu.SemaphoreType.DMA,
            pltpu.SemaphoreType.DMA,
            pltpu.SemaphoreType.DMA,
            pltpu.SemaphoreType.DMA,
        ],
    )
    def gather_kernel(w_hbm, i_hbm, o_hbm, idx_v, g_v, t_v, gs0, gs1, ws0, ws1):
        gsem = (gs0, gs1)
        wsem = (ws0, ws1)

        wid = lax.axis_index("s") * _NUM_CORES + lax.axis_index("c")
        base = wid * per_worker
        pltpu.sync_copy(i_hbm.at[0, pl.ds(base, per_worker)], idx_v)

        def start_gather(slot, c):
            pltpu.async_copy(
                w_hbm.at[idx_v.at[pl.ds(c * _CHUNK, _CHUNK)]],
                g_v.at[slot],
                gsem[slot],
            )

        def wait_gather(slot, c):
            pltpu.make_async_copy(
                w_hbm.at[idx_v.at[pl.ds(c * _CHUNK, _CHUNK)]],
                g_v.at[slot],
                gsem[slot],
            ).wait()

        def start_write(slot, c):
            @pl.loop(0, _CHUNK)
            def _(j):
                for k in range(dim // 16):
                    t_v[slot, j, 16 * k : 16 * k + 16] = g_v[
                        slot, j, 16 * k : 16 * k + 16
                    ]

            pltpu.async_copy(
                t_v.at[slot],
                o_hbm.at[pl.ds(base + c * _CHUNK, _CHUNK)],
                wsem[slot],
            )

        def wait_write(slot, c):
            pltpu.make_async_copy(
                t_v.at[slot],
                o_hbm.at[pl.ds(base + c * _CHUNK, _CHUNK)],
                wsem[slot],
            ).wait()

        start_gather(0, 0)
        start_gather(1, 1)

        @pl.loop(0, n_chunks, step=2)
        def _(c):
            for b in range(2):
                cc = c + b
                wait_gather(b, cc)
                start_write(b, cc)

                @pl.when(cc + 2 < n_chunks)
                def _():
                    wait_write(b, cc)
                    start_gather(b, cc + 2)

        wait_write(0, n_chunks - 2)
        wait_write(1, n_chunks - 1)

    out = gather_kernel(w_pad, flat_idx)
    return out.reshape(batch, seq, dim)


# speedup vs baseline: 1.7210x; 1.1104x over previous
import functools

import jax
import jax.numpy as jnp
from jax import lax
from jax.experimental import pallas as pl
from jax.experimental.pallas import tpu as pltpu
from jax.experimental.pallas import tpu_sc as plsc

_NUM_CORES = 2
_NUM_SUBCORES = 16
_NUM_WORKERS = _NUM_CORES * _NUM_SUBCORES
_CHUNK = 128
_PREP_BLK = 8192


def kernel(indices, weight):
    batch, seq = indices.shape
    vocab, dim = weight.shape
    n = batch * seq
    per_worker = n // _NUM_WORKERS
    n_chunks = per_worker // _CHUNK

    flat_idx = indices.reshape(1, n).astype(jnp.int32)

    eye = jnp.eye(dim, dtype=weight.dtype)

    def _prep_body(i_ref, e_ref, o_ref):
        x = i_ref[...]
        o_ref[:, :dim] = jax.lax.dot_general(
            x,
            e_ref[...],
            (((0,), (0,)), ((), ())),
            preferred_element_type=jnp.float32,
        )

    w_pad = pl.pallas_call(
        _prep_body,
        grid=((vocab + _PREP_BLK - 1) // _PREP_BLK,),
        in_specs=[
            pl.BlockSpec((dim, _PREP_BLK), lambda i: (0, i)),
            pl.BlockSpec((dim, dim), lambda i: (0, 0)),
        ],
        out_specs=pl.BlockSpec((_PREP_BLK, 128), lambda i: (i, 0)),
        out_shape=jax.ShapeDtypeStruct((vocab, 128), weight.dtype),
    )(weight.T, eye)

    mesh = plsc.VectorSubcoreMesh(core_axis_name="c", subcore_axis_name="s")

    @functools.partial(
        pl.kernel,
        out_type=jax.ShapeDtypeStruct((n, 128), weight.dtype),
        mesh=mesh,
        scratch_types=[
            pltpu.VMEM((per_worker,), jnp.int32),
            pltpu.VMEM((2, _CHUNK, 128), jnp.float32),
            pltpu.SemaphoreType.DMA,
            pltpu.SemaphoreType.DMA,
            pltpu.SemaphoreType.DMA,
            pltpu.SemaphoreType.DMA,
        ],
    )
    def gather_kernel(w_hbm, i_hbm, o_hbm, idx_v, g_v, gs0, gs1, ws0, ws1):
        gsem = (gs0, gs1)
        wsem = (ws0, ws1)

        wid = lax.axis_index("s") * _NUM_CORES + lax.axis_index("c")
        base = wid * per_worker
        pltpu.sync_copy(i_hbm.at[0, pl.ds(base, per_worker)], idx_v)

        def start_gather(slot, c):
            pltpu.async_copy(
                w_hbm.at[idx_v.at[pl.ds(c * _CHUNK, _CHUNK)]],
                g_v.at[slot],
                gsem[slot],
            )

        def wait_gather(slot, c):
            pltpu.make_async_copy(
                w_hbm.at[idx_v.at[pl.ds(c * _CHUNK, _CHUNK)]],
                g_v.at[slot],
                gsem[slot],
            ).wait()

        def start_write(slot, c):
            pltpu.async_copy(
                g_v.at[slot],
                o_hbm.at[pl.ds(base + c * _CHUNK, _CHUNK)],
                wsem[slot],
            )

        def wait_write(slot, c):
            pltpu.make_async_copy(
                g_v.at[slot],
                o_hbm.at[pl.ds(base + c * _CHUNK, _CHUNK)],
                wsem[slot],
            ).wait()

        start_gather(0, 0)
        start_gather(1, 1)

        @pl.loop(0, n_chunks, step=2)
        def _(c):
            for b in range(2):
                cc = c + b
                wait_gather(b, cc)
                start_write(b, cc)

                @pl.when(cc + 2 < n_chunks)
                def _():
                    wait_write(b, cc)
                    start_gather(b, cc + 2)

        wait_write(0, n_chunks - 2)
        wait_write(1, n_chunks - 1)

    rows = gather_kernel(w_pad, flat_idx)
    return rows[:, :dim].reshape(batch, seq, dim)
